# Initial kernel scaffold; baseline (speedup 1.0000x reference)
import jax
import jax.numpy as jnp
from jax.experimental import pallas as pl


def kernel(x, edge_index, Wl1, b1, Wr1, gamma1, beta1, Wl2, b2, Wr2):
    raise NotImplementedError("write your pallas kernel here")


# speedup vs baseline: 4.5260x; 4.5260x over previous
import functools

import jax
import jax.numpy as jnp
from jax import lax
from jax.experimental import pallas as pl
from jax.experimental.pallas import tpu as pltpu
from jax.experimental.pallas import tpu_sc as plsc

NC = 2
NS = 16
NW = NC * NS
CHUNK = 128


def _make_sc_aggregate(n_acc, d, ept, with_deg):
    rpt = n_acc // NS
    n_chunks = ept // CHUNK
    xr = 32
    nxb = rpt // xr
    mesh = plsc.VectorSubcoreMesh(core_axis_name="c", subcore_axis_name="s",
                                  num_cores=NC, num_subcores=NS)

    out_type = [jax.ShapeDtypeStruct((NC, n_acc, d), jnp.float32)]
    scratch = [
        pltpu.VMEM_SHARED((n_acc, d), jnp.float32),
        pltpu.VMEM((CHUNK,), jnp.int32),
        pltpu.VMEM((CHUNK,), jnp.int32),
        pltpu.VMEM((CHUNK, d), jnp.float32),
        pltpu.VMEM((xr, d), jnp.float32),
        pltpu.SemaphoreType.DMA,
    ]
    if with_deg:
        out_type.append(jax.ShapeDtypeStruct((NC, n_acc), jnp.float32))
        scratch += [
            pltpu.VMEM_SHARED((n_acc,), jnp.float32),
            pltpu.VMEM((CHUNK,), jnp.float32),
            pltpu.VMEM((rpt,), jnp.float32),
        ]

    def body(table, srcp, dstp, zeros_d, zeros_g, ones_g, *rest):
        if with_deg:
            (part, degp, acc, src_v, dst_v, rows_v, xbuf, sem,
             dega, ones_v, gbuf) = rest
        else:
            (part, acc, src_v, dst_v, rows_v, xbuf, sem) = rest
        c = lax.axis_index("c")
        s = lax.axis_index("s")
        wid = c * NS + s
        r0 = s * rpt

        pltpu.sync_copy(zeros_d, xbuf)
        for j in range(nxb):
            pltpu.sync_copy(xbuf, acc.at[pl.ds(r0 + j * xr, xr), :])
        if with_deg:
            pltpu.sync_copy(zeros_g, gbuf)
            pltpu.sync_copy(gbuf, dega.at[pl.ds(r0, rpt)])
            pltpu.sync_copy(ones_g, ones_v)
        plsc.subcore_barrier()

        def step(i, _):
            base = wid * ept + i * CHUNK
            pltpu.sync_copy(srcp.at[pl.ds(base, CHUNK)], src_v)
            pltpu.sync_copy(dstp.at[pl.ds(base, CHUNK)], dst_v)
            pltpu.async_copy(table.at[src_v], rows_v, sem).wait()
            pltpu.sync_copy(rows_v, acc.at[dst_v], add=True)
            if with_deg:
                pltpu.sync_copy(ones_v, dega.at[dst_v], add=True)
            return _

        lax.fori_loop(0, n_chunks, step, None)
        plsc.subcore_barrier()

        for j in range(nxb):
            sl = pl.ds(r0 + j * xr, xr)
            pltpu.sync_copy(acc.at[sl, :], xbuf)
            pltpu.sync_copy(xbuf, part.at[c, sl, :])
        if with_deg:
            pltpu.sync_copy(dega.at[pl.ds(r0, rpt)], gbuf)
            pltpu.sync_copy(gbuf, degp.at[c, pl.ds(r0, rpt)])

    return pl.kernel(body, out_type=tuple(out_type), mesh=mesh,
                     scratch_types=scratch)


def _lin2_body(x_ref, wl_ref, wr_ref, b_ref, y_ref, z_ref):
    x = x_ref[...]
    dn = (((1,), (1,)), ((), ()))
    y_ref[...] = lax.dot_general(x, wl_ref[...], dn,
                                 preferred_element_type=jnp.float32)
    z_ref[...] = lax.dot_general(x, wr_ref[...], dn,
                                 preferred_element_type=jnp.float32) \
        + b_ref[...][None, :]


def _relu_stats_body(n_valid, blk, p_ref, degp_ref, z_ref, h_ref, st_ref):
    i = pl.program_id(0)
    deg = jnp.maximum(degp_ref[0, :] + degp_ref[1, :], 1.0)
    h = jnp.maximum((p_ref[0] + p_ref[1]) / deg[:, None] + z_ref[...], 0.0)
    h_ref[...] = h

    @pl.when(i == 0)
    def _init():
        st_ref[...] = jnp.zeros_like(st_ref)

    row = i * blk + lax.broadcasted_iota(jnp.int32, (blk, 1), 0)
    hm = jnp.where(row < n_valid, h, 0.0)
    st_ref[0, :] += jnp.sum(hm, axis=0)
    st_ref[1, :] += jnp.sum(hm * hm, axis=0)


def _final_body(n_rows, h_ref, q_ref, degp_ref, st_ref, gamma_ref, beta_ref,
                wl_ref, wr_ref, b_ref, o_ref):
    inv_n = 1.0 / n_rows
    mu = st_ref[0, :] * inv_n
    var = st_ref[1, :] * inv_n - mu * mu
    sc = gamma_ref[...] * lax.rsqrt(var + 1e-5)
    t = beta_ref[...] - mu * sc
    deg = jnp.maximum(degp_ref[0, :] + degp_ref[1, :], 1.0)
    mn = (q_ref[0] + q_ref[1]) / deg[:, None]
    mn = mn * sc[None, :] + t[None, :]
    hn = h_ref[...] * sc[None, :] + t[None, :]
    dn = (((1,), (1,)), ((), ()))
    o = lax.dot_general(mn, wl_ref[...], dn,
                        preferred_element_type=jnp.float32) \
        + lax.dot_general(hn, wr_ref[...], dn,
                          preferred_element_type=jnp.float32) \
        + b_ref[...][None, :]
    m = jnp.max(o, axis=1, keepdims=True)
    lse = jnp.log(jnp.sum(jnp.exp(o - m), axis=1, keepdims=True))
    o_ref[...] = o - m - lse


def kernel(x, edge_index, Wl1, b1, Wr1, gamma1, beta1, Wl2, b2, Wr2):
    n, din = x.shape
    dh = Wl1.shape[0]
    dout = Wl2.shape[0]
    e = edge_index.shape[1]

    blk = 1024
    n_acc = ((n // blk) + 1) * blk
    grid = n_acc // blk
    ept = ((e // NW + CHUNK - 1) // CHUNK) * CHUNK
    epad = ept * NW
    rpt = n_acc // NS

    src = edge_index[0]
    dst = edge_index[1]
    pad = epad - e
    srcp = jnp.concatenate([src, jnp.zeros((pad,), jnp.int32)])
    dstp = jnp.concatenate([dst, jnp.full((pad,), n, jnp.int32)])
    xp = jnp.concatenate([x, jnp.zeros((n_acc - n, din), jnp.float32)])
    zeros_d = jnp.zeros((32, dh), jnp.float32)
    zeros_g = jnp.zeros((rpt,), jnp.float32)
    ones_g = jnp.ones((CHUNK,), jnp.float32)

    f32 = jnp.float32
    row_spec = pl.BlockSpec((blk, dh), lambda i: (i, 0))
    par_spec = pl.BlockSpec((NC, blk, dh), lambda i: (0, i, 0))
    deg_spec = pl.BlockSpec((NC, blk), lambda i: (0, i))

    y1, z1 = pl.pallas_call(
        _lin2_body,
        grid=(grid,),
        in_specs=[pl.BlockSpec((blk, din), lambda i: (i, 0)),
                  pl.BlockSpec((dh, din), lambda i: (0, 0)),
                  pl.BlockSpec((dh, din), lambda i: (0, 0)),
                  pl.BlockSpec((dh,), lambda i: (0,))],
        out_specs=[row_spec, row_spec],
        out_shape=[jax.ShapeDtypeStruct((n_acc, dh), f32),
                   jax.ShapeDtypeStruct((n_acc, dh), f32)],
    )(xp, Wl1, Wr1, b1)

    agg1 = _make_sc_aggregate(n_acc, dh, ept, with_deg=True)
    part1, degp = agg1(y1, srcp, dstp, zeros_d, zeros_g, ones_g)

    h, stats = pl.pallas_call(
        functools.partial(_relu_stats_body, n, blk),
        grid=(grid,),
        in_specs=[par_spec, deg_spec, row_spec],
        out_specs=[row_spec, pl.BlockSpec((2, dh), lambda i: (0, 0))],
        out_shape=[jax.ShapeDtypeStruct((n_acc, dh), f32),
                   jax.ShapeDtypeStruct((2, dh), f32)],
    )(part1, degp, z1)

    agg2 = _make_sc_aggregate(n_acc, dh, ept, with_deg=False)
    (part2,) = agg2(h, srcp, dstp, zeros_d, zeros_g, ones_g)

    out = pl.pallas_call(
        functools.partial(_final_body, float(n)),
        grid=(grid,),
        in_specs=[row_spec, par_spec, deg_spec,
                  pl.BlockSpec((2, dh), lambda i: (0, 0)),
                  pl.BlockSpec((dh,), lambda i: (0,)),
                  pl.BlockSpec((dh,), lambda i: (0,)),
                  pl.BlockSpec((dout, dh), lambda i: (0, 0)),
                  pl.BlockSpec((dout, dh), lambda i: (0, 0)),
                  pl.BlockSpec((dout,), lambda i: (0,))],
        out_specs=pl.BlockSpec((blk, dout), lambda i: (i, 0)),
        out_shape=jax.ShapeDtypeStruct((n_acc, dout), f32),
    )(h, part2, degp, stats, gamma1, beta1, Wl2, Wr2, b2)
    return out[:n]
